# baseline (speedup 1.0000x reference)
import jax
import jax.numpy as jnp
from jax.experimental import pallas as pl
from jax.experimental.pallas import tpu as pltpu

_N, _D, _H, _C = 10000, 128, 128, 2
_TILE = 1000


def _mask_kernel(x_ref, w1_ref, b1_ref, a_ref, w2_ref, b2_ref, wc_ref,
                 bc_ref, gu_ref, o_ref):
    h = jnp.dot(x_ref[...], w1_ref[...], preferred_element_type=jnp.float32)
    h = h + b1_ref[...]
    a = a_ref[0, 0]
    h = jnp.where(h >= 0.0, h, a * h)
    h = jnp.dot(h, w2_ref[...], preferred_element_type=jnp.float32)
    h = h + b2_ref[...]
    s = jnp.dot(h, wc_ref[...], preferred_element_type=jnp.float32)
    s = s + bc_ref[...]
    g = -jnp.log(-jnp.log(gu_ref[...]))
    s = s + g
    m = jnp.max(s, axis=-1, keepdims=True)
    e = jnp.exp(s - m)
    y = e / jnp.sum(e, axis=-1, keepdims=True)
    hard1 = (y[:, 1:2] > y[:, 0:1]).astype(jnp.float32)
    o_ref[...] = jnp.concatenate([1.0 - hard1, hard1], axis=-1)


def kernel(x, edge_index, W1, b1, prelu_a, W2, b2, Wc, bc, gumbel_u):
    del edge_index
    b1r = b1.reshape(1, _H)
    b2r = b2.reshape(1, _H)
    bcr = bc.reshape(1, _C)
    ar = prelu_a.reshape(1, 1)
    grid = (_N // _TILE,)
    fixed = lambda i: (0, 0)
    return pl.pallas_call(
        _mask_kernel,
        grid=grid,
        in_specs=[
            pl.BlockSpec((_TILE, _D), lambda i: (i, 0)),
            pl.BlockSpec((_D, _H), fixed),
            pl.BlockSpec((1, _H), fixed),
            pl.BlockSpec((1, 1), fixed),
            pl.BlockSpec((_H, _H), fixed),
            pl.BlockSpec((1, _H), fixed),
            pl.BlockSpec((_H, _C), fixed),
            pl.BlockSpec((1, _C), fixed),
            pl.BlockSpec((_TILE, _C), lambda i: (i, 0)),
        ],
        out_specs=pl.BlockSpec((_TILE, _C), lambda i: (i, 0)),
        out_shape=jax.ShapeDtypeStruct((_N, _C), jnp.float32),
        compiler_params=pltpu.CompilerParams(
            dimension_semantics=("parallel",)),
    )(x, W1, b1r, ar, W2, b2r, Wc, bcr, gumbel_u)


# speedup vs baseline: 1.8703x; 1.8703x over previous
import jax
import jax.numpy as jnp
from jax.experimental import pallas as pl
from jax.experimental.pallas import tpu as pltpu

_N, _D, _H, _C = 10000, 128, 128, 2
_TILE = 2000


def _mask_kernel(x_ref, w1t_ref, b1_ref, a_ref, w2t_ref, b2_ref, wct_ref,
                 bc_ref, gut_ref, o_ref):
    xt = x_ref[0].T
    h = jnp.dot(w1t_ref[...], xt, preferred_element_type=jnp.float32)
    h = h + b1_ref[...]
    a = a_ref[0, 0]
    h = jnp.where(h >= 0.0, h, a * h)
    h = jnp.dot(w2t_ref[...], h, preferred_element_type=jnp.float32)
    h = h + b2_ref[...]
    s = jnp.dot(wct_ref[...], h, preferred_element_type=jnp.float32)
    s = s + bc_ref[...]
    g = -jnp.log(-jnp.log(gut_ref[0]))
    s = s + g
    m = jnp.max(s, axis=0, keepdims=True)
    e = jnp.exp(s - m)
    y = e / jnp.sum(e, axis=0, keepdims=True)
    hard1 = (y[1:2, :] > y[0:1, :]).astype(jnp.float32)
    o_ref[0] = jnp.concatenate([1.0 - hard1, hard1], axis=0)


def kernel(x, edge_index, W1, b1, prelu_a, W2, b2, Wc, bc, gumbel_u):
    del edge_index
    n_blk = _N // _TILE
    x3 = x.reshape(n_blk, _TILE, _D)
    gut3 = gumbel_u.reshape(n_blk, _TILE, _C).transpose(0, 2, 1)
    w1t = W1.T
    w2t = W2.T
    wct = Wc.T
    b1c = b1.reshape(_H, 1)
    b2c = b2.reshape(_H, 1)
    bcc = bc.reshape(_C, 1)
    ar = prelu_a.reshape(1, 1)
    fixed = lambda i: (0, 0)
    ot = pl.pallas_call(
        _mask_kernel,
        grid=(n_blk,),
        in_specs=[
            pl.BlockSpec((1, _TILE, _D), lambda i: (i, 0, 0)),
            pl.BlockSpec((_H, _D), fixed),
            pl.BlockSpec((_H, 1), fixed),
            pl.BlockSpec((1, 1), fixed),
            pl.BlockSpec((_H, _H), fixed),
            pl.BlockSpec((_H, 1), fixed),
            pl.BlockSpec((_C, _H), fixed),
            pl.BlockSpec((_C, 1), fixed),
            pl.BlockSpec((1, _C, _TILE), lambda i: (i, 0, 0)),
        ],
        out_specs=pl.BlockSpec((1, _C, _TILE), lambda i: (i, 0, 0)),
        out_shape=jax.ShapeDtypeStruct((n_blk, _C, _TILE), jnp.float32),
    )(x3, w1t, b1c, ar, w2t, b2c, wct, bcc, gut3)
    return ot.transpose(0, 2, 1).reshape(_N, _C)
